# baseline (speedup 1.0000x reference)
import functools

import jax
import jax.numpy as jnp
from jax.experimental import pallas as pl
from jax.experimental.pallas import tpu as pltpu

_EPS_NORM = 1e-8
_EPS_LN = 1e-5
_F32 = jnp.float32

_TE = 1024
_WIN = 256
_TN_NORM = 2048
_TN_NODE = 1024
_VMEM = 48 * 1024 * 1024
_VMEM_BIG = 56 * 1024 * 1024


def _ru(x, m):
    return ((x + m - 1) // m) * m


def _pad_rows(x, target, axis=0):
    pad = target - x.shape[axis]
    if pad <= 0:
        return x
    widths = [(0, 0)] * x.ndim
    widths[axis] = (0, pad)
    return jnp.pad(x, widths)


def _ln_scalar(s, g, b):
    mu = jnp.mean(s, axis=-1, keepdims=True)
    var = jnp.mean((s - mu) ** 2, axis=-1, keepdims=True)
    return (s - mu) * jax.lax.rsqrt(var + _EPS_LN) * g + b


def _ln_vec(v3):
    sq = v3[0] * v3[0] + v3[1] * v3[1] + v3[2] * v3[2]
    inv = 1.0 / jnp.sqrt(jnp.mean(sq, axis=-1, keepdims=True) + _EPS_NORM)
    return v3 * inv[None]


def _gcp(s, v3, wd, wss, wsv, bs, wu, wg, bg, act):
    rows = s.shape[0]
    v_in = v3.shape[2]
    h = wd.shape[1]
    v_out = wu.shape[1]
    hv = jnp.dot(v3.reshape(3 * rows, v_in), wd, preferred_element_type=_F32)
    hv3 = hv.reshape(3, rows, h)
    vn = jnp.sqrt(hv3[0] * hv3[0] + hv3[1] * hv3[1] + hv3[2] * hv3[2]
                  + _EPS_NORM)
    s_pre = (jnp.dot(s, wss, preferred_element_type=_F32)
             + jnp.dot(vn, wsv, preferred_element_type=_F32) + bs)
    g_in = jax.nn.silu(s_pre) if act else s_pre
    gate = jax.nn.sigmoid(jnp.dot(g_in, wg, preferred_element_type=_F32) + bg)
    vo = (jnp.dot(hv, wu, preferred_element_type=_F32)
          .reshape(3, rows, v_out) * gate[None])
    so = jax.nn.silu(s_pre) if act else s_pre
    return so, vo


def _dotT(a, b):
    return jax.lax.dot_general(a, b, (((0,), (0,)), ((), ())),
                               preferred_element_type=_F32)


def _norm_body(s_ref, v_ref, g_ref, b_ref, s0_ref, v0_ref):
    s0_ref[...] = _ln_scalar(s_ref[...], g_ref[...], b_ref[...])
    v0_ref[...] = _ln_vec(v_ref[...])


def _msg_scatter_body(base_ref, nwin_ref, col_ref, srow_ref, vrow_ref,
                      ef_ref, tab_ref,
                      wd0, wss0, wsv0, bs0, wu0, wg0, bg0,
                      wd1, wss1, wsv1, bs1, wu1, wg1, bg1,
                      out_ref, gcol_ref, *, half, win, e_s, e_v):
    p = pl.program_id(0)
    e = pl.program_id(1)

    @pl.when(e == 0)
    def _():
        out_ref[...] = jnp.zeros_like(out_ref)

    blk = p * half + e
    base = base_ref[blk]
    nw = nwin_ref[blk]
    colv = col_ref[0]
    te = colv.shape[-1]
    nv = wu0.shape[1]

    gcol_ref[...] = jnp.zeros_like(gcol_ref)

    def gather_win(w, carry):
        lo = pl.multiple_of(base + w * win, 8)
        ids = jax.lax.broadcasted_iota(jnp.int32, (win, te), 0) + lo
        oh = (ids == colv).astype(jnp.bfloat16)
        gcol_ref[...] += _dotT(oh, tab_ref[pl.ds(lo, win), :]).astype(
            jnp.bfloat16)
        return carry

    jax.lax.fori_loop(0, nw, gather_win, 0)

    srow = srow_ref[...]
    vrow = vrow_ref[...]
    ns = srow.shape[1]
    scol = gcol_ref[:, :ns]
    vcol = gcol_ref[:, ns:]
    ef = ef_ref[...]
    ev = [ef[:, e_s + k * e_v: e_s + (k + 1) * e_v] for k in range(3)]
    wd = wd0[...]
    hv3 = jnp.stack(
        [jnp.dot(vrow[:, k * nv:(k + 1) * nv], wd[:nv],
                 preferred_element_type=_F32)
         + jnp.dot(ev[k], wd[nv:nv + e_v], preferred_element_type=_F32)
         + jnp.dot(vcol[:, k * nv:(k + 1) * nv], wd[nv + e_v:],
                   preferred_element_type=_F32)
         for k in range(3)], axis=0)
    vn = jnp.sqrt(hv3[0] * hv3[0] + hv3[1] * hv3[1] + hv3[2] * hv3[2]
                  + _EPS_NORM)
    wss = wss0[...]
    s_pre = (jnp.dot(srow, wss[:ns], preferred_element_type=_F32)
             + jnp.dot(ef[:, :e_s], wss[ns:ns + e_s],
                       preferred_element_type=_F32)
             + jnp.dot(scol, wss[ns + e_s:],
                       preferred_element_type=_F32)
             + jnp.dot(vn.astype(jnp.bfloat16), wsv0[...],
                       preferred_element_type=_F32)
             + bs0[...])
    g_in = jax.nn.silu(s_pre)
    gate = jax.nn.sigmoid(jnp.dot(g_in, wg0[...], preferred_element_type=_F32)
                          + bg0[...])
    m_v = (jnp.dot(hv3.reshape(3 * te, -1), wu0[...],
                   preferred_element_type=_F32).reshape(3, te, nv)
           * gate[None])
    m_s = g_in

    n_s, n_v = _gcp(m_s, m_v, wd1[...], wss1[...], wsv1[...], bs1[...],
                    wu1[...], wg1[...], bg1[...], act=False)
    s = m_s + n_s
    v = m_v + n_v
    msg = jnp.concatenate(
        [s, v[0], v[1], v[2], jnp.ones((te, 1), _F32)], axis=-1)

    def scatter_win(w, carry):
        lo = pl.multiple_of(base + w * win, 8)
        ids = jax.lax.broadcasted_iota(jnp.int32, (win, te), 0) + lo
        oh = (ids == colv).astype(_F32)
        out_ref[0, pl.ds(lo, win), :] += jnp.dot(
            oh, msg, preferred_element_type=_F32)
        return carry

    jax.lax.fori_loop(0, nw, scatter_win, 0)


def _node_body(s0_ref, v0_ref, agg_ref, g1, b1,
               wda, wssa, wsva, bsa, wua, wga, bga,
               wdb, wssb, wsvb, bsb, wub, wgb, bgb, out_ref):
    ns = s0_ref.shape[1]
    nv = v0_ref.shape[2]
    tot = jnp.sum(agg_ref[...], axis=0)
    cnt = tot[:, ns + 3 * nv:]
    agg = tot[:, :ns + 3 * nv] * (1.0 / jnp.maximum(cnt, 1.0))
    s1 = s0_ref[...] + agg[:, :ns]
    v1 = v0_ref[...] + jnp.stack(
        [agg[:, ns + k * nv: ns + (k + 1) * nv] for k in range(3)], axis=0)
    s2 = _ln_scalar(s1, g1[...], b1[...])
    v2 = _ln_vec(v1)
    h_s, h_v = _gcp(s2, v2, wda[...], wssa[...], wsva[...], bsa[...],
                    wua[...], wga[...], bga[...], act=True)
    o_s, o_v = _gcp(h_s, h_v, wdb[...], wssb[...], wsvb[...], bsb[...],
                    wub[...], wgb[...], bgb[...], act=False)
    out_s = s2 + o_s
    out_v = v2 + o_v
    out_ref[...] = jnp.concatenate(
        [out_s, out_v[0], out_v[1], out_v[2]], axis=-1)


def kernel(node_s, node_v, edge_s, edge_v, edge_index, frames,
           msg0_Wd, msg0_Wss, msg0_Wsv, msg0_bs, msg0_Wu, msg0_Wg, msg0_bg,
           msg1_Wd, msg1_Wss, msg1_Wsv, msg1_bs, msg1_Wu, msg1_Wg, msg1_bg,
           norm0_gamma, norm0_beta, norm1_gamma, norm1_beta,
           ff0_Wd, ff0_Wss, ff0_Wsv, ff0_bs, ff0_Wu, ff0_Wg, ff0_bg,
           ff1_Wd, ff1_Wss, ff1_Wsv, ff1_bs, ff1_Wu, ff1_Wg, ff1_bg):
    del frames
    N, ns = node_s.shape
    nv = node_v.shape[1]
    E = edge_s.shape[0]
    M = ns + 3 * nv
    MO = M + 1

    node_s = node_s.astype(_F32)
    edge_s = edge_s.astype(_F32)
    nv3 = jnp.transpose(node_v.astype(_F32), (2, 0, 1))
    ev3 = jnp.transpose(edge_v.astype(_F32), (2, 0, 1))
    row = edge_index[0].astype(jnp.int32)
    col = edge_index[1].astype(jnp.int32)

    te = min(_TE, _ru(E, 128))
    n_eblk = _ru(E, te) // te
    if n_eblk % 2:
        n_eblk += 1
    e_pad = n_eblk * te
    half = n_eblk // 2

    tn = min(_TN_NODE, _ru(N, 8))
    n_pad = _ru(N, tn)
    tn_norm = min(_TN_NORM, _ru(N, 8))
    n_pad = _ru(n_pad, tn_norm)
    n_out = max(n_pad, _ru(N, 8) + _WIN)

    s0, v0 = pl.pallas_call(
        _norm_body,
        grid=(n_pad // tn_norm,),
        in_specs=[pl.BlockSpec((tn_norm, ns), lambda i: (i, 0)),
                  pl.BlockSpec((3, tn_norm, nv), lambda i: (0, i, 0)),
                  pl.BlockSpec(norm0_gamma.shape, lambda i: (0, 0)),
                  pl.BlockSpec(norm0_beta.shape, lambda i: (0, 0))],
        out_specs=(pl.BlockSpec((tn_norm, ns), lambda i: (i, 0)),
                   pl.BlockSpec((3, tn_norm, nv), lambda i: (0, i, 0))),
        out_shape=(jax.ShapeDtypeStruct((n_pad, ns), _F32),
                   jax.ShapeDtypeStruct((3, n_pad, nv), _F32)),
        compiler_params=pltpu.CompilerParams(
            dimension_semantics=("parallel",), vmem_limit_bytes=_VMEM),
    )(_pad_rows(node_s, n_pad), _pad_rows(nv3, n_pad, axis=1),
      norm0_gamma, norm0_beta)

    perm = jnp.argsort(col)
    row_s = row[perm]
    col_s = col[perm]
    bf16 = jnp.bfloat16
    vpack = jnp.concatenate([v0[0], v0[1], v0[2]], axis=-1)
    s0b = s0.astype(bf16)
    vpackb = vpack.astype(bf16)
    srow = _pad_rows(s0b[row_s], e_pad)
    vrow = _pad_rows(vpackb[row_s], e_pad)
    efeat = _pad_rows(jnp.concatenate(
        [edge_s, ev3[0], ev3[1], ev3[2]], axis=-1).astype(bf16)[perm], e_pad)
    tab_big = _pad_rows(jnp.concatenate([s0b, vpackb], axis=-1),
                        n_out)

    big = jnp.int32(2 ** 30)
    col_pad = jnp.concatenate(
        [col_s, jnp.full((e_pad - E,), big, jnp.int32)])
    cs = col_pad.reshape(n_eblk, te)
    cmin = cs.min(axis=1)
    cmax = jnp.max(jnp.where(cs < N, cs, -1), axis=1)
    base = jnp.clip((cmin // 8) * 8, 0, max(N - 8, 0)).astype(jnp.int32)
    nwin = jnp.maximum(0, (cmax - base + _WIN) // _WIN).astype(jnp.int32)

    w0 = (msg0_Wd.astype(bf16), msg0_Wss.astype(bf16), msg0_Wsv.astype(bf16),
          msg0_bs, msg0_Wu, msg0_Wg, msg0_bg)
    w1 = (msg1_Wd, msg1_Wss, msg1_Wsv, msg1_bs, msg1_Wu, msg1_Wg, msg1_bg)
    full = lambda a: pl.BlockSpec(a.shape, lambda p, e, *_: (0,) * a.ndim)
    grid_spec = pltpu.PrefetchScalarGridSpec(
        num_scalar_prefetch=2,
        grid=(2, half),
        in_specs=[pl.BlockSpec((1, 1, te),
                               lambda p, e, *_: (p * half + e, 0, 0)),
                  pl.BlockSpec((te, ns), lambda p, e, *_: (p * half + e, 0)),
                  pl.BlockSpec((te, 3 * nv),
                               lambda p, e, *_: (p * half + e, 0)),
                  pl.BlockSpec((te, efeat.shape[1]),
                               lambda p, e, *_: (p * half + e, 0)),
                  full(tab_big)]
                 + [full(w) for w in w0] + [full(w) for w in w1],
        out_specs=pl.BlockSpec((1, n_out, MO), lambda p, e, *_: (p, 0, 0)),
        scratch_shapes=[pltpu.VMEM((te, M), jnp.bfloat16)],
    )
    agg2 = pl.pallas_call(
        functools.partial(_msg_scatter_body, half=half, win=_WIN,
                          e_s=edge_s.shape[1], e_v=ev3.shape[2]),
        grid_spec=grid_spec,
        out_shape=jax.ShapeDtypeStruct((2, n_out, MO), _F32),
        compiler_params=pltpu.CompilerParams(
            dimension_semantics=("parallel", "arbitrary"),
            vmem_limit_bytes=_VMEM_BIG),
    )(base, nwin, col_pad.reshape(n_eblk, 1, te),
      srow, vrow, efeat, tab_big, *w0, *w1)

    wa = (ff0_Wd, ff0_Wss, ff0_Wsv, ff0_bs, ff0_Wu, ff0_Wg, ff0_bg)
    wb = (ff1_Wd, ff1_Wss, ff1_Wsv, ff1_bs, ff1_Wu, ff1_Wg, ff1_bg)
    fulln = lambda a: pl.BlockSpec(a.shape, lambda i: (0,) * a.ndim)
    out_flat = pl.pallas_call(
        _node_body,
        grid=(n_pad // tn,),
        in_specs=[pl.BlockSpec((tn, ns), lambda i: (i, 0)),
                  pl.BlockSpec((3, tn, nv), lambda i: (0, i, 0)),
                  pl.BlockSpec((2, tn, MO), lambda i: (0, i, 0)),
                  fulln(norm1_gamma), fulln(norm1_beta)]
                 + [fulln(w) for w in wa] + [fulln(w) for w in wb],
        out_specs=pl.BlockSpec((tn, M), lambda i: (i, 0)),
        out_shape=jax.ShapeDtypeStruct((n_pad, M), _F32),
        compiler_params=pltpu.CompilerParams(
            dimension_semantics=("parallel",), vmem_limit_bytes=_VMEM),
    )(s0, v0, agg2, norm1_gamma, norm1_beta, *wa, *wb)

    out_s = out_flat[:N, :ns]
    out_v = out_flat[:N, ns:].reshape(N, 3, nv).transpose(0, 2, 1)
    return out_s, out_v


# speedup vs baseline: 2.2953x; 2.2953x over previous
import functools

import jax
import jax.numpy as jnp
from jax.experimental import pallas as pl
from jax.experimental.pallas import tpu as pltpu

_EPS_NORM = 1e-8
_EPS_LN = 1e-5
_F32 = jnp.float32

_TE = 1024
_WIN = 256
_TN_NORM = 2048
_TN_NODE = 1024
_VMEM = 48 * 1024 * 1024
_VMEM_BIG = 56 * 1024 * 1024


def _ru(x, m):
    return ((x + m - 1) // m) * m


def _pad_rows(x, target, axis=0):
    pad = target - x.shape[axis]
    if pad <= 0:
        return x
    widths = [(0, 0)] * x.ndim
    widths[axis] = (0, pad)
    return jnp.pad(x, widths)


def _ln_scalar(s, g, b):
    mu = jnp.mean(s, axis=-1, keepdims=True)
    var = jnp.mean((s - mu) ** 2, axis=-1, keepdims=True)
    return (s - mu) * jax.lax.rsqrt(var + _EPS_LN) * g + b


def _ln_vec(v3):
    sq = v3[0] * v3[0] + v3[1] * v3[1] + v3[2] * v3[2]
    inv = 1.0 / jnp.sqrt(jnp.mean(sq, axis=-1, keepdims=True) + _EPS_NORM)
    return v3 * inv[None]


def _bdot(a, b):
    return jnp.dot(a.astype(jnp.bfloat16), b.astype(jnp.bfloat16),
                   preferred_element_type=_F32)


def _gcp(s, v3, wd, wss, wsv, bs, wu, wg, bg, act):
    rows = s.shape[0]
    v_in = v3.shape[2]
    h = wd.shape[1]
    v_out = wu.shape[1]
    hv = _bdot(v3.reshape(3 * rows, v_in), wd)
    hv3 = hv.reshape(3, rows, h)
    vn = jnp.sqrt(hv3[0] * hv3[0] + hv3[1] * hv3[1] + hv3[2] * hv3[2]
                  + _EPS_NORM)
    s_pre = _bdot(s, wss) + _bdot(vn, wsv) + bs
    g_in = jax.nn.silu(s_pre) if act else s_pre
    gate = jax.nn.sigmoid(_bdot(g_in, wg) + bg)
    vo = _bdot(hv, wu).reshape(3, rows, v_out) * gate[None]
    so = jax.nn.silu(s_pre) if act else s_pre
    return so, vo


def _dotT(a, b):
    return jax.lax.dot_general(a, b, (((0,), (0,)), ((), ())),
                               preferred_element_type=_F32)


def _norm_body(s_ref, v_ref, g_ref, b_ref, s0_ref, v0_ref):
    s0_ref[...] = _ln_scalar(s_ref[...], g_ref[...], b_ref[...])
    v0_ref[...] = _ln_vec(v_ref[...])


def _msg_scatter_body(base_ref, nwin_ref, col_ref, srow_ref, vrow_ref,
                      ef_ref, tab_ref,
                      wd0, wss0, wsv0, bs0, wu0, wg0, bg0,
                      wd1, wss1, wsv1, bs1, wu1, wg1, bg1,
                      out_ref, gcol_ref, *, half, win, e_s, e_v):
    p = pl.program_id(0)
    e = pl.program_id(1)

    @pl.when(e == 0)
    def _():
        out_ref[...] = jnp.zeros_like(out_ref)

    blk = p * half + e
    base = base_ref[blk]
    nw = nwin_ref[blk]
    colv = col_ref[0]
    te = colv.shape[-1]
    nv = wu0.shape[1]

    gcol_ref[...] = jnp.zeros_like(gcol_ref)

    def gather_win(w, carry):
        lo = pl.multiple_of(base + w * win, 8)
        ids = jax.lax.broadcasted_iota(jnp.int32, (win, te), 0) + lo
        oh = (ids == colv).astype(jnp.bfloat16)
        gcol_ref[...] += _dotT(oh, tab_ref[pl.ds(lo, win), :])
        return carry

    jax.lax.fori_loop(0, nw, gather_win, 0)

    srow = srow_ref[...]
    vrow = vrow_ref[...]
    ns = srow.shape[1]
    scol = gcol_ref[:, :ns]
    vcol = gcol_ref[:, ns:]
    ef = ef_ref[...]
    ev = [ef[:, e_s + k * e_v: e_s + (k + 1) * e_v] for k in range(3)]
    wd = wd0[...]
    hv3 = jnp.stack(
        [_bdot(vrow[:, k * nv:(k + 1) * nv], wd[:nv])
         + _bdot(ev[k], wd[nv:nv + e_v])
         + _bdot(vcol[:, k * nv:(k + 1) * nv], wd[nv + e_v:])
         for k in range(3)], axis=0)
    vn = jnp.sqrt(hv3[0] * hv3[0] + hv3[1] * hv3[1] + hv3[2] * hv3[2]
                  + _EPS_NORM)
    wss = wss0[...]
    s_pre = (_bdot(srow, wss[:ns])
             + _bdot(ef[:, :e_s], wss[ns:ns + e_s])
             + _bdot(scol, wss[ns + e_s:])
             + _bdot(vn, wsv0[...])
             + bs0[...])
    g_in = jax.nn.silu(s_pre)
    gate = jax.nn.sigmoid(_bdot(g_in, wg0[...]) + bg0[...])
    m_v = _bdot(hv3.reshape(3 * te, -1), wu0[...]).reshape(3, te, nv) \
        * gate[None]
    m_s = g_in

    n_s, n_v = _gcp(m_s, m_v, wd1[...], wss1[...], wsv1[...], bs1[...],
                    wu1[...], wg1[...], bg1[...], act=False)
    s = m_s + n_s
    v = m_v + n_v
    msg = jnp.concatenate(
        [s, v[0], v[1], v[2], jnp.ones((te, 1), _F32)],
        axis=-1).astype(jnp.bfloat16)

    def scatter_win(w, carry):
        lo = pl.multiple_of(base + w * win, 8)
        ids = jax.lax.broadcasted_iota(jnp.int32, (win, te), 0) + lo
        oh = (ids == colv).astype(jnp.bfloat16)
        out_ref[0, pl.ds(lo, win), :] += jnp.dot(
            oh, msg, preferred_element_type=_F32)
        return carry

    jax.lax.fori_loop(0, nw, scatter_win, 0)


def _node_body(s0_ref, v0_ref, agg_ref, g1, b1,
               wda, wssa, wsva, bsa, wua, wga, bga,
               wdb, wssb, wsvb, bsb, wub, wgb, bgb, out_ref):
    ns = s0_ref.shape[1]
    nv = v0_ref.shape[2]
    tot = jnp.sum(agg_ref[...], axis=0)
    cnt = tot[:, ns + 3 * nv:]
    agg = tot[:, :ns + 3 * nv] * (1.0 / jnp.maximum(cnt, 1.0))
    s1 = s0_ref[...] + agg[:, :ns]
    v1 = v0_ref[...] + jnp.stack(
        [agg[:, ns + k * nv: ns + (k + 1) * nv] for k in range(3)], axis=0)
    s2 = _ln_scalar(s1, g1[...], b1[...])
    v2 = _ln_vec(v1)
    h_s, h_v = _gcp(s2, v2, wda[...], wssa[...], wsva[...], bsa[...],
                    wua[...], wga[...], bga[...], act=True)
    o_s, o_v = _gcp(h_s, h_v, wdb[...], wssb[...], wsvb[...], bsb[...],
                    wub[...], wgb[...], bgb[...], act=False)
    out_s = s2 + o_s
    out_v = v2 + o_v
    out_ref[...] = jnp.concatenate(
        [out_s, out_v[0], out_v[1], out_v[2]], axis=-1)


def kernel(node_s, node_v, edge_s, edge_v, edge_index, frames,
           msg0_Wd, msg0_Wss, msg0_Wsv, msg0_bs, msg0_Wu, msg0_Wg, msg0_bg,
           msg1_Wd, msg1_Wss, msg1_Wsv, msg1_bs, msg1_Wu, msg1_Wg, msg1_bg,
           norm0_gamma, norm0_beta, norm1_gamma, norm1_beta,
           ff0_Wd, ff0_Wss, ff0_Wsv, ff0_bs, ff0_Wu, ff0_Wg, ff0_bg,
           ff1_Wd, ff1_Wss, ff1_Wsv, ff1_bs, ff1_Wu, ff1_Wg, ff1_bg):
    del frames
    N, ns = node_s.shape
    nv = node_v.shape[1]
    E = edge_s.shape[0]
    M = ns + 3 * nv
    MO = M + 1

    node_s = node_s.astype(_F32)
    edge_s = edge_s.astype(_F32)
    nv3 = jnp.transpose(node_v.astype(_F32), (2, 0, 1))
    ev3 = jnp.transpose(edge_v.astype(_F32), (2, 0, 1))
    row = edge_index[0].astype(jnp.int32)
    col = edge_index[1].astype(jnp.int32)

    te = min(_TE, _ru(E, 128))
    n_eblk = _ru(E, te) // te
    if n_eblk % 2:
        n_eblk += 1
    e_pad = n_eblk * te
    half = n_eblk // 2

    tn = min(_TN_NODE, _ru(N, 8))
    n_pad = _ru(N, tn)
    tn_norm = min(_TN_NORM, _ru(N, 8))
    n_pad = _ru(n_pad, tn_norm)
    n_out = max(n_pad, _ru(N, 8) + _WIN)

    s0, v0 = pl.pallas_call(
        _norm_body,
        grid=(n_pad // tn_norm,),
        in_specs=[pl.BlockSpec((tn_norm, ns), lambda i: (i, 0)),
                  pl.BlockSpec((3, tn_norm, nv), lambda i: (0, i, 0)),
                  pl.BlockSpec(norm0_gamma.shape, lambda i: (0, 0)),
                  pl.BlockSpec(norm0_beta.shape, lambda i: (0, 0))],
        out_specs=(pl.BlockSpec((tn_norm, ns), lambda i: (i, 0)),
                   pl.BlockSpec((3, tn_norm, nv), lambda i: (0, i, 0))),
        out_shape=(jax.ShapeDtypeStruct((n_pad, ns), _F32),
                   jax.ShapeDtypeStruct((3, n_pad, nv), _F32)),
        compiler_params=pltpu.CompilerParams(
            dimension_semantics=("parallel",), vmem_limit_bytes=_VMEM),
    )(_pad_rows(node_s, n_pad), _pad_rows(nv3, n_pad, axis=1),
      norm0_gamma, norm0_beta)

    perm = jnp.argsort(col)
    row_s = row[perm]
    col_s = col[perm]
    bf16 = jnp.bfloat16
    vpack = jnp.concatenate([v0[0], v0[1], v0[2]], axis=-1)
    s0b = s0.astype(bf16)
    vpackb = vpack.astype(bf16)
    srow = _pad_rows(s0[row_s], e_pad)
    vrow = _pad_rows(vpack[row_s], e_pad)
    efeat = _pad_rows(jnp.concatenate(
        [edge_s, ev3[0], ev3[1], ev3[2]], axis=-1)[perm], e_pad)
    tab_big = _pad_rows(jnp.concatenate([s0b, vpackb], axis=-1),
                        n_out)

    big = jnp.int32(2 ** 30)
    col_pad = jnp.concatenate(
        [col_s, jnp.full((e_pad - E,), big, jnp.int32)])
    cs = col_pad.reshape(n_eblk, te)
    cmin = cs.min(axis=1)
    cmax = jnp.max(jnp.where(cs < N, cs, -1), axis=1)
    base = jnp.clip((cmin // 8) * 8, 0, max(N - 8, 0)).astype(jnp.int32)
    nwin = jnp.maximum(0, (cmax - base + _WIN) // _WIN).astype(jnp.int32)

    w0 = (msg0_Wd, msg0_Wss, msg0_Wsv, msg0_bs, msg0_Wu, msg0_Wg, msg0_bg)
    w1 = (msg1_Wd, msg1_Wss, msg1_Wsv, msg1_bs, msg1_Wu, msg1_Wg, msg1_bg)
    full = lambda a: pl.BlockSpec(a.shape, lambda p, e, *_: (0,) * a.ndim)
    grid_spec = pltpu.PrefetchScalarGridSpec(
        num_scalar_prefetch=2,
        grid=(2, half),
        in_specs=[pl.BlockSpec((1, 1, te),
                               lambda p, e, *_: (p * half + e, 0, 0)),
                  pl.BlockSpec((te, ns), lambda p, e, *_: (p * half + e, 0)),
                  pl.BlockSpec((te, 3 * nv),
                               lambda p, e, *_: (p * half + e, 0)),
                  pl.BlockSpec((te, efeat.shape[1]),
                               lambda p, e, *_: (p * half + e, 0)),
                  full(tab_big)]
                 + [full(w) for w in w0] + [full(w) for w in w1],
        out_specs=pl.BlockSpec((1, n_out, MO), lambda p, e, *_: (p, 0, 0)),
        scratch_shapes=[pltpu.VMEM((te, M), _F32)],
    )
    agg2 = pl.pallas_call(
        functools.partial(_msg_scatter_body, half=half, win=_WIN,
                          e_s=edge_s.shape[1], e_v=ev3.shape[2]),
        grid_spec=grid_spec,
        out_shape=jax.ShapeDtypeStruct((2, n_out, MO), _F32),
        compiler_params=pltpu.CompilerParams(
            dimension_semantics=("parallel", "arbitrary"),
            vmem_limit_bytes=_VMEM_BIG),
    )(base, nwin, col_pad.reshape(n_eblk, 1, te),
      srow, vrow, efeat, tab_big, *w0, *w1)

    wa = (ff0_Wd, ff0_Wss, ff0_Wsv, ff0_bs, ff0_Wu, ff0_Wg, ff0_bg)
    wb = (ff1_Wd, ff1_Wss, ff1_Wsv, ff1_bs, ff1_Wu, ff1_Wg, ff1_bg)
    fulln = lambda a: pl.BlockSpec(a.shape, lambda i: (0,) * a.ndim)
    out_flat = pl.pallas_call(
        _node_body,
        grid=(n_pad // tn,),
        in_specs=[pl.BlockSpec((tn, ns), lambda i: (i, 0)),
                  pl.BlockSpec((3, tn, nv), lambda i: (0, i, 0)),
                  pl.BlockSpec((2, tn, MO), lambda i: (0, i, 0)),
                  fulln(norm1_gamma), fulln(norm1_beta)]
                 + [fulln(w) for w in wa] + [fulln(w) for w in wb],
        out_specs=pl.BlockSpec((tn, M), lambda i: (i, 0)),
        out_shape=jax.ShapeDtypeStruct((n_pad, M), _F32),
        compiler_params=pltpu.CompilerParams(
            dimension_semantics=("parallel",), vmem_limit_bytes=_VMEM),
    )(s0, v0, agg2, norm1_gamma, norm1_beta, *wa, *wb)

    out_s = out_flat[:N, :ns]
    out_v = out_flat[:N, ns:].reshape(N, 3, nv).transpose(0, 2, 1)
    return out_s, out_v
